# baseline (speedup 1.0000x reference)
import functools

import jax
import jax.numpy as jnp
from jax import lax
from jax.experimental import pallas as pl
from jax.experimental.pallas import tpu as pltpu
from jax.experimental.pallas import tpu_sc as plsc

N = 10000
D = 128
DH = D // 2
E = 320000
NC = 2
NS = 16
EW = E // NS
CHUNK = 125
NCHUNK = EW // CHUNK
NBUF = 8
ROWS_MAIN = 624
TAIL_BASE = ROWS_MAIN * NS
TAIL = N - TAIL_BASE


def _spmm_body(g_hbm, col_hbm, row_hbm, zeros_hbm, out_hbm,
               colv, ridx, *rest):
    bufs = rest[:NBUF]
    accum = rest[NBUF]
    gsems = rest[NBUF + 1:2 * NBUF + 1]
    rsems = rest[2 * NBUF + 1:3 * NBUF + 1]
    ssems = rest[3 * NBUF + 1:]
    cid = lax.axis_index("c")
    sid = lax.axis_index("s")
    pltpu.sync_copy(col_hbm.at[sid], colv)
    coff = cid * DH

    def gather_copy(chunk, b):
        return pltpu.make_async_copy(
            g_hbm.at[cid].at[colv.at[chunk]], bufs[b], gsems[b])

    def ridx_copy(chunk, b):
        return pltpu.make_async_copy(row_hbm.at[sid, chunk], ridx.at[b],
                                     rsems[b])

    for b in range(NBUF):
        gather_copy(b, b).start()
        ridx_copy(b, b).start()

    pltpu.sync_copy(zeros_hbm.at[pl.ds(0, ROWS_MAIN)],
                    accum.at[pl.ds(sid * ROWS_MAIN, ROWS_MAIN)])

    @pl.when(sid == NS - 1)
    def _zero_tail():
        pltpu.sync_copy(zeros_hbm.at[pl.ds(0, TAIL)],
                        accum.at[pl.ds(TAIL_BASE, TAIL)])

    plsc.subcore_barrier()

    def scatter_wait(b):
        pltpu.make_async_copy(bufs[b], accum.at[ridx.at[b]], ssems[b]).wait()

    def outer_step(jj, carry):
        for b in range(NBUF):
            chunk = jj * NBUF + b
            gather_copy(chunk, b).wait()
            ridx_copy(chunk, b).wait()
            pltpu.async_copy(bufs[b], accum.at[ridx.at[b]], ssems[b],
                             add=True)
            b2 = (b + NBUF - 3) % NBUF
            nxt = chunk + NBUF - 3

            @pl.when(jnp.logical_and(nxt >= NBUF, nxt < NCHUNK))
            def _prefetch():
                scatter_wait(b2)
                gather_copy(nxt, b2).start()
                ridx_copy(nxt, b2).start()

        return carry

    lax.fori_loop(0, NCHUNK // NBUF, outer_step, 0)
    for b in range(NBUF):
        scatter_wait(b)
    plsc.subcore_barrier()
    pltpu.sync_copy(accum.at[pl.ds(sid * ROWS_MAIN, ROWS_MAIN)],
                    out_hbm.at[pl.ds(sid * ROWS_MAIN, ROWS_MAIN),
                               pl.ds(coff, DH)])

    @pl.when(sid == NS - 1)
    def _export_tail():
        pltpu.sync_copy(accum.at[pl.ds(TAIL_BASE, TAIL)],
                        out_hbm.at[pl.ds(TAIL_BASE, TAIL), pl.ds(coff, DH)])


_spmm = pl.kernel(
    _spmm_body,
    mesh=plsc.VectorSubcoreMesh(core_axis_name="c", subcore_axis_name="s"),
    compiler_params=pltpu.CompilerParams(use_tc_tiling_on_sc=False),
    out_type=jax.ShapeDtypeStruct((N, D), jnp.float32),
    scratch_types=(
        [pltpu.VMEM((NCHUNK, CHUNK), jnp.int32),
         pltpu.VMEM((NBUF, CHUNK), jnp.int32)]
        + [pltpu.VMEM((CHUNK, DH), jnp.float32)] * NBUF
        + [pltpu.VMEM_SHARED((N, DH), jnp.float32)]
        + [pltpu.SemaphoreType.DMA] * (3 * NBUF)
    ),
)

RB = 1000
GRID = N // RB


def _combine_body(include_input, p, h, outh, outg, csum):
    hn = jnp.sign(p[...] + h[...])
    outh[...] = hn
    g = jnp.roll(hn, 1, axis=1)
    outg[...] = jnp.stack([g[:, :DH], g[:, DH:]], axis=0)
    part = jnp.sum(hn, axis=0, keepdims=True)
    if include_input:
        part = part + jnp.sum(h[...], axis=0, keepdims=True)

    @pl.when(pl.program_id(0) == 0)
    def _init():
        csum[...] = part

    @pl.when(pl.program_id(0) != 0)
    def _acc():
        csum[...] = csum[...] + part


def _make_combine(include_input):
    return pl.pallas_call(
        functools.partial(_combine_body, include_input),
        grid=(GRID,),
        in_specs=[pl.BlockSpec((RB, D), lambda i: (i, 0))] * 2,
        out_specs=[pl.BlockSpec((RB, D), lambda i: (i, 0)),
                   pl.BlockSpec((NC, RB, DH), lambda i: (0, i, 0)),
                   pl.BlockSpec((1, D), lambda i: (0, 0))],
        out_shape=[jax.ShapeDtypeStruct((N, D), jnp.float32),
                   jax.ShapeDtypeStruct((NC, N, DH), jnp.float32),
                   jax.ShapeDtypeStruct((1, D), jnp.float32)],
    )


_combine_first = _make_combine(True)
_combine_rest = _make_combine(False)


def _prep_body(x, outg):
    g = jnp.roll(x[...], 1, axis=1)
    outg[...] = jnp.stack([g[:, :DH], g[:, DH:]], axis=0)


_prep = pl.pallas_call(
    _prep_body,
    grid=(GRID,),
    in_specs=[pl.BlockSpec((RB, D), lambda i: (i, 0))],
    out_specs=pl.BlockSpec((NC, RB, DH), lambda i: (0, i, 0)),
    out_shape=jax.ShapeDtypeStruct((NC, N, DH), jnp.float32),
)


def kernel(x, edge_index):
    row = edge_index[0].reshape(NS, NCHUNK, CHUNK)
    col = edge_index[1].reshape(NS, NCHUNK, CHUNK)
    zeros = jnp.zeros((ROWS_MAIN, DH), jnp.float32)
    g = _prep(x)
    h = x
    total = None
    for layer in range(3):
        p = _spmm(g, col, row, zeros)
        combine = _combine_first if layer == 0 else _combine_rest
        h, g, csum = combine(p, h)
        total = csum if total is None else total + csum
    return total.reshape(D)


# speedup vs baseline: 1.0186x; 1.0186x over previous
import functools

import jax
import jax.numpy as jnp
from jax import lax
from jax.experimental import pallas as pl
from jax.experimental.pallas import tpu as pltpu
from jax.experimental.pallas import tpu_sc as plsc

N = 10000
D = 128
DH = D // 2
E = 320000
NC = 2
NS = 16
EW = E // NS
CHUNK = 125
NCHUNK = EW // CHUNK
NBUF = 8
ROWS_MAIN = 624
TAIL_BASE = ROWS_MAIN * NS
TAIL = N - TAIL_BASE


def _spmm_body(g_hbm, col_hbm, row_hbm, zeros_hbm, out_hbm,
               colv, ridx, *rest):
    bufs = rest[:NBUF]
    accum = rest[NBUF]
    gsems = rest[NBUF + 1:2 * NBUF + 1]
    rsems = rest[2 * NBUF + 1:3 * NBUF + 1]
    ssems = rest[3 * NBUF + 1:]
    cid = lax.axis_index("c")
    sid = lax.axis_index("s")
    pltpu.sync_copy(col_hbm.at[sid], colv)
    coff = cid * DH

    def gather_copy(chunk, b):
        return pltpu.make_async_copy(
            g_hbm.at[cid].at[colv.at[chunk]], bufs[b], gsems[b])

    def ridx_copy(chunk, b):
        return pltpu.make_async_copy(row_hbm.at[sid, chunk], ridx.at[b],
                                     rsems[b])

    for b in range(NBUF):
        gather_copy(b, b).start()
        ridx_copy(b, b).start()

    pltpu.sync_copy(zeros_hbm.at[pl.ds(0, ROWS_MAIN)],
                    accum.at[pl.ds(sid * ROWS_MAIN, ROWS_MAIN)])

    @pl.when(sid == NS - 1)
    def _zero_tail():
        pltpu.sync_copy(zeros_hbm.at[pl.ds(0, TAIL)],
                        accum.at[pl.ds(TAIL_BASE, TAIL)])

    plsc.subcore_barrier()

    def scatter_wait(b):
        pltpu.make_async_copy(bufs[b], accum.at[ridx.at[b]], ssems[b]).wait()

    def outer_step(jj, carry):
        for b in range(NBUF):
            chunk = jj * NBUF + b
            gather_copy(chunk, b).wait()
            ridx_copy(chunk, b).wait()
            pltpu.async_copy(bufs[b], accum.at[ridx.at[b]], ssems[b],
                             add=True)
            b2 = (b + NBUF - 2) % NBUF
            nxt = chunk + NBUF - 2

            @pl.when(jnp.logical_and(nxt >= NBUF, nxt < NCHUNK))
            def _prefetch():
                scatter_wait(b2)
                gather_copy(nxt, b2).start()
                ridx_copy(nxt, b2).start()

        return carry

    lax.fori_loop(0, NCHUNK // NBUF, outer_step, 0)
    for b in range(NBUF):
        scatter_wait(b)
    plsc.subcore_barrier()
    pltpu.sync_copy(accum.at[pl.ds(sid * ROWS_MAIN, ROWS_MAIN)],
                    out_hbm.at[pl.ds(sid * ROWS_MAIN, ROWS_MAIN),
                               pl.ds(coff, DH)])

    @pl.when(sid == NS - 1)
    def _export_tail():
        pltpu.sync_copy(accum.at[pl.ds(TAIL_BASE, TAIL)],
                        out_hbm.at[pl.ds(TAIL_BASE, TAIL), pl.ds(coff, DH)])


_spmm = pl.kernel(
    _spmm_body,
    mesh=plsc.VectorSubcoreMesh(core_axis_name="c", subcore_axis_name="s"),
    compiler_params=pltpu.CompilerParams(use_tc_tiling_on_sc=False),
    out_type=jax.ShapeDtypeStruct((N, D), jnp.float32),
    scratch_types=(
        [pltpu.VMEM((NCHUNK, CHUNK), jnp.int32),
         pltpu.VMEM((NBUF, CHUNK), jnp.int32)]
        + [pltpu.VMEM((CHUNK, DH), jnp.float32)] * NBUF
        + [pltpu.VMEM_SHARED((N, DH), jnp.float32)]
        + [pltpu.SemaphoreType.DMA] * (3 * NBUF)
    ),
)

RB = 1000
GRID = N // RB


def _combine_body(include_input, p, h, outh, outg, csum):
    hn = jnp.sign(p[...] + h[...])
    outh[...] = hn
    g = jnp.roll(hn, 1, axis=1)
    outg[...] = jnp.stack([g[:, :DH], g[:, DH:]], axis=0)
    part = jnp.sum(hn, axis=0, keepdims=True)
    if include_input:
        part = part + jnp.sum(h[...], axis=0, keepdims=True)

    @pl.when(pl.program_id(0) == 0)
    def _init():
        csum[...] = part

    @pl.when(pl.program_id(0) != 0)
    def _acc():
        csum[...] = csum[...] + part


def _make_combine(include_input):
    return pl.pallas_call(
        functools.partial(_combine_body, include_input),
        grid=(GRID,),
        in_specs=[pl.BlockSpec((RB, D), lambda i: (i, 0))] * 2,
        out_specs=[pl.BlockSpec((RB, D), lambda i: (i, 0)),
                   pl.BlockSpec((NC, RB, DH), lambda i: (0, i, 0)),
                   pl.BlockSpec((1, D), lambda i: (0, 0))],
        out_shape=[jax.ShapeDtypeStruct((N, D), jnp.float32),
                   jax.ShapeDtypeStruct((NC, N, DH), jnp.float32),
                   jax.ShapeDtypeStruct((1, D), jnp.float32)],
    )


_combine_first = _make_combine(True)
_combine_rest = _make_combine(False)


def _combine_last_body(p, h, csum):
    hn = jnp.sign(p[...] + h[...])
    part = jnp.sum(hn, axis=0, keepdims=True)

    @pl.when(pl.program_id(0) == 0)
    def _init():
        csum[...] = part

    @pl.when(pl.program_id(0) != 0)
    def _acc():
        csum[...] = csum[...] + part


_combine_last = pl.pallas_call(
    _combine_last_body,
    grid=(GRID,),
    in_specs=[pl.BlockSpec((RB, D), lambda i: (i, 0))] * 2,
    out_specs=pl.BlockSpec((1, D), lambda i: (0, 0)),
    out_shape=jax.ShapeDtypeStruct((1, D), jnp.float32),
)


def _prep_body(x, outg):
    g = jnp.roll(x[...], 1, axis=1)
    outg[...] = jnp.stack([g[:, :DH], g[:, DH:]], axis=0)


_prep = pl.pallas_call(
    _prep_body,
    grid=(GRID,),
    in_specs=[pl.BlockSpec((RB, D), lambda i: (i, 0))],
    out_specs=pl.BlockSpec((NC, RB, DH), lambda i: (0, i, 0)),
    out_shape=jax.ShapeDtypeStruct((NC, N, DH), jnp.float32),
)


def kernel(x, edge_index):
    row = edge_index[0].reshape(NS, NCHUNK, CHUNK)
    col = edge_index[1].reshape(NS, NCHUNK, CHUNK)
    zeros = jnp.zeros((ROWS_MAIN, DH), jnp.float32)
    g = _prep(x)
    h = x
    total = None
    for layer in range(3):
        p = _spmm(g, col, row, zeros)
        if layer < 2:
            combine = _combine_first if layer == 0 else _combine_rest
            h, g, csum = combine(p, h)
        else:
            csum = _combine_last(p, h)
        total = csum if total is None else total + csum
    return total.reshape(D)


# speedup vs baseline: 1.0425x; 1.0235x over previous
import functools

import jax
import jax.numpy as jnp
from jax import lax
from jax.experimental import pallas as pl
from jax.experimental.pallas import tpu as pltpu
from jax.experimental.pallas import tpu_sc as plsc

N = 10000
D = 128
DH = D // 2
E = 320000
NC = 2
NS = 16
EW = E // NS
CHUNK = 125
NCHUNK = EW // CHUNK
NBUF = 8
ROWS_MAIN = 624
TAIL_BASE = ROWS_MAIN * NS
TAIL = N - TAIL_BASE


def _spmm_body(g_hbm, col_hbm, row_hbm, zeros_hbm, out_hbm,
               colv, ridx, *rest):
    bufs = rest[:NBUF]
    accum = rest[NBUF]
    gsems = rest[NBUF + 1:2 * NBUF + 1]
    rsems = rest[2 * NBUF + 1:3 * NBUF + 1]
    ssems = rest[3 * NBUF + 1:]
    cid = lax.axis_index("c")
    sid = lax.axis_index("s")
    pltpu.sync_copy(col_hbm.at[sid], colv)
    coff = cid * DH

    def gather_copy(chunk, b):
        return pltpu.make_async_copy(
            g_hbm.at[cid].at[colv.at[chunk]], bufs[b], gsems[b])

    def ridx_copy(chunk, b):
        return pltpu.make_async_copy(row_hbm.at[sid, chunk], ridx.at[b],
                                     rsems[b])

    for b in range(NBUF):
        gather_copy(b, b).start()
        ridx_copy(b, b).start()

    pltpu.sync_copy(zeros_hbm.at[pl.ds(0, ROWS_MAIN)],
                    accum.at[pl.ds(sid * ROWS_MAIN, ROWS_MAIN)])

    @pl.when(sid == NS - 1)
    def _zero_tail():
        pltpu.sync_copy(zeros_hbm.at[pl.ds(0, TAIL)],
                        accum.at[pl.ds(TAIL_BASE, TAIL)])

    plsc.subcore_barrier()

    def scatter_wait(b):
        pltpu.make_async_copy(bufs[b], accum.at[ridx.at[b]], ssems[b]).wait()

    def outer_step(jj, carry):
        for b in range(NBUF):
            chunk = jj * NBUF + b
            gather_copy(chunk, b).wait()
            ridx_copy(chunk, b).wait()
            pltpu.async_copy(bufs[b], accum.at[ridx.at[b]], ssems[b],
                             add=True)
            b2 = (b + NBUF - 2) % NBUF
            nxt = chunk + NBUF - 2

            @pl.when(jnp.logical_and(nxt >= NBUF, nxt < NCHUNK))
            def _prefetch():
                scatter_wait(b2)
                gather_copy(nxt, b2).start()
                ridx_copy(nxt, b2).start()

        return carry

    lax.fori_loop(0, NCHUNK // NBUF, outer_step, 0)
    for b in range(NBUF):
        scatter_wait(b)
    plsc.subcore_barrier()
    pltpu.sync_copy(accum.at[pl.ds(sid * ROWS_MAIN, ROWS_MAIN)],
                    out_hbm.at[pl.ds(sid * ROWS_MAIN, ROWS_MAIN),
                               pl.ds(coff, DH)])

    @pl.when(sid == NS - 1)
    def _export_tail():
        pltpu.sync_copy(accum.at[pl.ds(TAIL_BASE, TAIL)],
                        out_hbm.at[pl.ds(TAIL_BASE, TAIL), pl.ds(coff, DH)])


_spmm = pl.kernel(
    _spmm_body,
    mesh=plsc.VectorSubcoreMesh(core_axis_name="c", subcore_axis_name="s"),
    compiler_params=pltpu.CompilerParams(use_tc_tiling_on_sc=False),
    out_type=jax.ShapeDtypeStruct((N, D), jnp.float32),
    scratch_types=(
        [pltpu.VMEM((NCHUNK, CHUNK), jnp.int32),
         pltpu.VMEM((NBUF, CHUNK), jnp.int32)]
        + [pltpu.VMEM((CHUNK, DH), jnp.float32)] * NBUF
        + [pltpu.VMEM_SHARED((N, DH), jnp.float32)]
        + [pltpu.SemaphoreType.DMA] * (3 * NBUF)
    ),
)

RB = 2000
GRID = N // RB


def _combine_body(include_input, p, h, outh, outg, csum):
    hn = jnp.sign(p[...] + h[...])
    outh[...] = hn
    g = jnp.roll(hn, 1, axis=1)
    outg[...] = jnp.stack([g[:, :DH], g[:, DH:]], axis=0)
    part = jnp.sum(hn, axis=0, keepdims=True)
    if include_input:
        part = part + jnp.sum(h[...], axis=0, keepdims=True)

    @pl.when(pl.program_id(0) == 0)
    def _init():
        csum[...] = part

    @pl.when(pl.program_id(0) != 0)
    def _acc():
        csum[...] = csum[...] + part


def _make_combine(include_input):
    return pl.pallas_call(
        functools.partial(_combine_body, include_input),
        grid=(GRID,),
        in_specs=[pl.BlockSpec((RB, D), lambda i: (i, 0))] * 2,
        out_specs=[pl.BlockSpec((RB, D), lambda i: (i, 0)),
                   pl.BlockSpec((NC, RB, DH), lambda i: (0, i, 0)),
                   pl.BlockSpec((1, D), lambda i: (0, 0))],
        out_shape=[jax.ShapeDtypeStruct((N, D), jnp.float32),
                   jax.ShapeDtypeStruct((NC, N, DH), jnp.float32),
                   jax.ShapeDtypeStruct((1, D), jnp.float32)],
    )


_combine_first = _make_combine(True)
_combine_rest = _make_combine(False)


def _combine_last_body(p, h, csum):
    hn = jnp.sign(p[...] + h[...])
    part = jnp.sum(hn, axis=0, keepdims=True)

    @pl.when(pl.program_id(0) == 0)
    def _init():
        csum[...] = part

    @pl.when(pl.program_id(0) != 0)
    def _acc():
        csum[...] = csum[...] + part


_combine_last = pl.pallas_call(
    _combine_last_body,
    grid=(GRID,),
    in_specs=[pl.BlockSpec((RB, D), lambda i: (i, 0))] * 2,
    out_specs=pl.BlockSpec((1, D), lambda i: (0, 0)),
    out_shape=jax.ShapeDtypeStruct((1, D), jnp.float32),
)


def _prep_body(x, outg):
    g = jnp.roll(x[...], 1, axis=1)
    outg[...] = jnp.stack([g[:, :DH], g[:, DH:]], axis=0)


_prep = pl.pallas_call(
    _prep_body,
    grid=(GRID,),
    in_specs=[pl.BlockSpec((RB, D), lambda i: (i, 0))],
    out_specs=pl.BlockSpec((NC, RB, DH), lambda i: (0, i, 0)),
    out_shape=jax.ShapeDtypeStruct((NC, N, DH), jnp.float32),
)


def kernel(x, edge_index):
    row = edge_index[0].reshape(NS, NCHUNK, CHUNK)
    col = edge_index[1].reshape(NS, NCHUNK, CHUNK)
    zeros = jnp.zeros((ROWS_MAIN, DH), jnp.float32)
    g = _prep(x)
    h = x
    total = None
    for layer in range(3):
        p = _spmm(g, col, row, zeros)
        if layer < 2:
            combine = _combine_first if layer == 0 else _combine_rest
            h, g, csum = combine(p, h)
        else:
            csum = _combine_last(p, h)
        total = csum if total is None else total + csum
    return total.reshape(D)


# speedup vs baseline: 1.0606x; 1.0174x over previous
import functools

import jax
import jax.numpy as jnp
from jax import lax
from jax.experimental import pallas as pl
from jax.experimental.pallas import tpu as pltpu
from jax.experimental.pallas import tpu_sc as plsc

N = 10000
D = 128
DH = D // 2
E = 320000
NC = 2
NS = 16
EW = E // NS
CHUNK = 125
NCHUNK = EW // CHUNK
NBUF = 8
ROWS_MAIN = 624
TAIL_BASE = ROWS_MAIN * NS
TAIL = N - TAIL_BASE


def _spmm_body(g_hbm, col_hbm, row_hbm, zeros_hbm, out_hbm,
               colv, ridx, *rest):
    bufs = rest[:NBUF]
    accum = rest[NBUF]
    gsems = rest[NBUF + 1:2 * NBUF + 1]
    rsems = rest[2 * NBUF + 1:3 * NBUF + 1]
    ssems = rest[3 * NBUF + 1:]
    cid = lax.axis_index("c")
    sid = lax.axis_index("s")
    pltpu.sync_copy(col_hbm.at[sid], colv)
    coff = cid * DH

    def gather_copy(chunk, b):
        return pltpu.make_async_copy(
            g_hbm.at[cid].at[colv.at[chunk]], bufs[b], gsems[b])

    def ridx_copy(chunk, b):
        return pltpu.make_async_copy(row_hbm.at[sid, chunk], ridx.at[b],
                                     rsems[b])

    for b in range(NBUF):
        gather_copy(b, b).start()
        ridx_copy(b, b).start()

    pltpu.sync_copy(zeros_hbm.at[pl.ds(0, ROWS_MAIN)],
                    accum.at[pl.ds(sid * ROWS_MAIN, ROWS_MAIN)])

    @pl.when(sid == NS - 1)
    def _zero_tail():
        pltpu.sync_copy(zeros_hbm.at[pl.ds(0, TAIL)],
                        accum.at[pl.ds(TAIL_BASE, TAIL)])

    plsc.subcore_barrier()

    def scatter_wait(b):
        pltpu.make_async_copy(bufs[b], accum.at[ridx.at[b]], ssems[b]).wait()

    def outer_step(jj, carry):
        for b in range(NBUF):
            chunk = jj * NBUF + b
            gather_copy(chunk, b).wait()
            ridx_copy(chunk, b).wait()
            pltpu.async_copy(bufs[b], accum.at[ridx.at[b]], ssems[b],
                             add=True)
            b2 = (b + NBUF - 2) % NBUF
            nxt = chunk + NBUF - 2

            @pl.when(jnp.logical_and(nxt >= NBUF, nxt < NCHUNK))
            def _prefetch():
                scatter_wait(b2)
                gather_copy(nxt, b2).start()
                ridx_copy(nxt, b2).start()

        return carry

    lax.fori_loop(0, NCHUNK // NBUF, outer_step, 0)
    for b in range(NBUF):
        scatter_wait(b)
    plsc.subcore_barrier()
    pltpu.sync_copy(accum.at[pl.ds(sid * ROWS_MAIN, ROWS_MAIN)],
                    out_hbm.at[pl.ds(sid * ROWS_MAIN, ROWS_MAIN),
                               pl.ds(coff, DH)])

    @pl.when(sid == NS - 1)
    def _export_tail():
        pltpu.sync_copy(accum.at[pl.ds(TAIL_BASE, TAIL)],
                        out_hbm.at[pl.ds(TAIL_BASE, TAIL), pl.ds(coff, DH)])


_spmm = pl.kernel(
    _spmm_body,
    mesh=plsc.VectorSubcoreMesh(core_axis_name="c", subcore_axis_name="s"),
    compiler_params=pltpu.CompilerParams(use_tc_tiling_on_sc=False),
    out_type=jax.ShapeDtypeStruct((N, D), jnp.float32),
    scratch_types=(
        [pltpu.VMEM((NCHUNK, CHUNK), jnp.int32),
         pltpu.VMEM((NBUF, CHUNK), jnp.int32)]
        + [pltpu.VMEM((CHUNK, DH), jnp.float32)] * NBUF
        + [pltpu.VMEM_SHARED((N, DH), jnp.float32)]
        + [pltpu.SemaphoreType.DMA] * (3 * NBUF)
    ),
)

RB = 5000
GRID = N // RB


def _combine_body(include_input, p, h, outh, outg, csum):
    hn = jnp.sign(p[...] + h[...])
    outh[...] = hn
    g = jnp.roll(hn, 1, axis=1)
    outg[...] = jnp.stack([g[:, :DH], g[:, DH:]], axis=0)
    part = jnp.sum(hn, axis=0, keepdims=True)
    if include_input:
        part = part + jnp.sum(h[...], axis=0, keepdims=True)

    @pl.when(pl.program_id(0) == 0)
    def _init():
        csum[...] = part

    @pl.when(pl.program_id(0) != 0)
    def _acc():
        csum[...] = csum[...] + part


def _make_combine(include_input):
    return pl.pallas_call(
        functools.partial(_combine_body, include_input),
        grid=(GRID,),
        in_specs=[pl.BlockSpec((RB, D), lambda i: (i, 0))] * 2,
        out_specs=[pl.BlockSpec((RB, D), lambda i: (i, 0)),
                   pl.BlockSpec((NC, RB, DH), lambda i: (0, i, 0)),
                   pl.BlockSpec((1, D), lambda i: (0, 0))],
        out_shape=[jax.ShapeDtypeStruct((N, D), jnp.float32),
                   jax.ShapeDtypeStruct((NC, N, DH), jnp.float32),
                   jax.ShapeDtypeStruct((1, D), jnp.float32)],
    )


_combine_first = _make_combine(True)
_combine_rest = _make_combine(False)


def _combine_last_body(p, h, csum):
    hn = jnp.sign(p[...] + h[...])
    part = jnp.sum(hn, axis=0, keepdims=True)

    @pl.when(pl.program_id(0) == 0)
    def _init():
        csum[...] = part

    @pl.when(pl.program_id(0) != 0)
    def _acc():
        csum[...] = csum[...] + part


_combine_last = pl.pallas_call(
    _combine_last_body,
    grid=(GRID,),
    in_specs=[pl.BlockSpec((RB, D), lambda i: (i, 0))] * 2,
    out_specs=pl.BlockSpec((1, D), lambda i: (0, 0)),
    out_shape=jax.ShapeDtypeStruct((1, D), jnp.float32),
)


def _prep_body(x, outg):
    g = jnp.roll(x[...], 1, axis=1)
    outg[...] = jnp.stack([g[:, :DH], g[:, DH:]], axis=0)


_prep = pl.pallas_call(
    _prep_body,
    grid=(GRID,),
    in_specs=[pl.BlockSpec((RB, D), lambda i: (i, 0))],
    out_specs=pl.BlockSpec((NC, RB, DH), lambda i: (0, i, 0)),
    out_shape=jax.ShapeDtypeStruct((NC, N, DH), jnp.float32),
)


def kernel(x, edge_index):
    row = edge_index[0].reshape(NS, NCHUNK, CHUNK)
    col = edge_index[1].reshape(NS, NCHUNK, CHUNK)
    zeros = jnp.zeros((ROWS_MAIN, DH), jnp.float32)
    g = _prep(x)
    h = x
    total = None
    for layer in range(3):
        p = _spmm(g, col, row, zeros)
        if layer < 2:
            combine = _combine_first if layer == 0 else _combine_rest
            h, g, csum = combine(p, h)
        else:
            csum = _combine_last(p, h)
        total = csum if total is None else total + csum
    return total.reshape(D)
